# baseline (speedup 1.0000x reference)
import functools

import jax
import jax.numpy as jnp
from jax import lax
from jax.experimental import pallas as pl
from jax.experimental.pallas import tpu as pltpu
from jax.experimental.pallas import tpu_sc as plsc

NC = 2
NS = 16
NW = NC * NS
VOCAB = 1000000
EMB = 64
SCALE = 8.0
VB = 256
NFULL = VOCAB // VB
NJ_BASE = NFULL // NW
TAIL = VOCAB - NFULL * VB
CHUNK = 128


def _k1_format_table(lut_t, tail_flat):
    mesh = plsc.VectorSubcoreMesh(core_axis_name="c", subcore_axis_name="s")

    @functools.partial(
        pl.kernel,
        out_type=jax.ShapeDtypeStruct((VOCAB * EMB // 128, 128), jnp.float32),
        mesh=mesh,
        scratch_types=[
            pltpu.VMEM((EMB, VB), jnp.float32),
            pltpu.VMEM((EMB, VB), jnp.float32),
            pltpu.VMEM((VB // 2, 129), jnp.float32),
            pltpu.VMEM((VB // 2, 129), jnp.float32),
            pltpu.VMEM((TAIL * EMB // 128, 128), jnp.float32),
            pltpu.SemaphoreType.DMA,
            pltpu.SemaphoreType.DMA,
            pltpu.SemaphoreType.DMA,
            pltpu.SemaphoreType.DMA,
        ],
        compiler_params=pltpu.CompilerParams(
            use_tc_tiling_on_sc=True, needs_layout_passes=False),
    )
    def k(lut_hbm, tail_hbm, out_hbm, buf0, buf1, fb0, fb1, tbuf,
          si0, si1, so0, so1):
        wid = lax.axis_index("s") * NC + lax.axis_index("c")
        bufs = (buf0, buf1)
        fbs = (fb0, fb1)
        sis = (si0, si1)
        sos = (so0, so1)

        @pl.when(wid == 0)
        def _():
            pltpu.sync_copy(tail_hbm, tbuf)
            pltpu.sync_copy(
                tbuf, out_hbm.at[pl.ds(NFULL * VB * EMB // 128, TAIL * EMB // 128), :])

        n_j = NJ_BASE + jnp.where(wid < NFULL - NW * NJ_BASE, 1, 0)
        QR = VB // 2
        iota16 = jnp.arange(16, dtype=jnp.int32)
        q_ids = [(iota16 + 16 * mm) // 2 for mm in range(VB // 16)]

        def start_in(b, j):
            bid = wid + NW * j
            pltpu.async_copy(
                lut_hbm.at[:, pl.ds(bid * VB, VB)], bufs[b], sis[b])

        def wait_in(b):
            pltpu.make_async_copy(
                lut_hbm.at[:, pl.ds(0, VB)], bufs[b], sis[b]).wait()

        def start_out(b, j):
            bid = wid + NW * j
            pltpu.async_copy(
                fbs[b].at[:, pl.ds(0, 128)],
                out_hbm.at[pl.ds(bid * QR, QR), :], sos[b])

        def drain_out(b):
            pltpu.make_async_copy(
                fbs[b].at[:, pl.ds(0, 128)],
                out_hbm.at[pl.ds(0, QR), :], sos[b]).wait()

        def transpose(b):
            e8 = (iota16 % 2) * 8
            p64 = (iota16 % 2) * EMB

            @plsc.parallel_loop(0, EMB, unroll=8)
            def _(f):
                rv = (jnp.full((16,), f, dtype=jnp.int32) ^ e8) + p64
                for mm in range(VB // 16):
                    vals = bufs[b][f, pl.ds(16 * mm, 16)] * SCALE
                    plsc.store_scatter(fbs[b], [q_ids[mm], rv], vals)

        start_in(0, 0)
        start_in(1, 1)

        def outer(jj, carry):
            for b in range(2):
                j = jj * 2 + b
                wait_in(b)

                @pl.when(j >= 2)
                def _():
                    drain_out(b)

                transpose(b)
                start_out(b, j)

                @pl.when(j + 2 < n_j)
                def _():
                    start_in(b, j + 2)

            return carry

        lax.fori_loop(0, NJ_BASE // 2, outer, 0)

        @pl.when(n_j > NJ_BASE)
        def _():
            wait_in(0)
            drain_out(0)
            transpose(0)
            start_out(0, NJ_BASE)

        drain_out(0)
        drain_out(1)

    return k(lut_t, tail_flat)


def _k2_lookup(xtf, table):
    L5, FT, BT, FR, BC = 50, 8, 128, 8, 128
    n_chunks = L5 * BT // NW
    mesh = plsc.VectorSubcoreMesh(core_axis_name="c", subcore_axis_name="s")

    @functools.partial(
        pl.kernel,
        out_type=jax.ShapeDtypeStruct((L5, FT, BT, FR, BC), jnp.float32),
        mesh=mesh,
        scratch_types=[
            pltpu.VMEM((n_chunks * CHUNK,), jnp.int32),
            pltpu.VMEM((n_chunks * CHUNK,), jnp.int32),
            pltpu.VMEM((CHUNK, EMB), jnp.float32),
            pltpu.VMEM((CHUNK, EMB), jnp.float32),
            pltpu.VMEM((CHUNK, EMB), jnp.float32),
            pltpu.VMEM((CHUNK, EMB), jnp.float32),
            pltpu.VMEM((EMB, BC + 1), jnp.float32),
            pltpu.VMEM((EMB, BC + 1), jnp.float32),
            pltpu.VMEM((EMB, BC + 1), jnp.float32),
            pltpu.VMEM((EMB, BC + 1), jnp.float32),
            pltpu.SemaphoreType.DMA,
            pltpu.SemaphoreType.DMA,
            pltpu.SemaphoreType.DMA,
            pltpu.SemaphoreType.DMA,
            pltpu.SemaphoreType.DMA,
            pltpu.SemaphoreType.DMA,
            pltpu.SemaphoreType.DMA,
            pltpu.SemaphoreType.DMA,
        ],
        compiler_params=pltpu.CompilerParams(
            use_tc_tiling_on_sc=False, needs_layout_passes=False),
    )
    def k(x_hbm, t_hbm, out_hbm, idx_v, par8_v, r0, r1, r2, r3,
          t0, t1, t2, t3, sg0, sg1, sg2, sg3, ss0, ss1, ss2, ss3):
        wid = lax.axis_index("s") * NC + lax.axis_index("c")
        rows = (r0, r1, r2, r3)
        tbs = (t0, t1, t2, t3)
        sgs = (sg0, sg1, sg2, sg3)
        sss = (ss0, ss1, ss2, ss3)
        c0 = wid * n_chunks

        pltpu.sync_copy(x_hbm.at[pl.ds(c0 * CHUNK, n_chunks * CHUNK)], idx_v)

        @plsc.parallel_loop(0, n_chunks * CHUNK // 16, unroll=8)
        def _(i):
            sl = pl.ds(i * 16, 16)
            par8_v[sl] = (idx_v[sl] & 1) << 3

        row_ids = [jnp.arange(16, dtype=jnp.int32) + 16 * mm for mm in range(4)]

        def start_gather(b, c):
            pltpu.async_copy(
                t_hbm.at[idx_v.at[pl.ds(c * CHUNK, CHUNK)]], rows[b], sgs[b])

        def wait_gather(b):
            pltpu.make_async_copy(
                t_hbm.at[idx_v.at[pl.ds(0, CHUNK)]], rows[b], sgs[b]).wait()

        def start_store(b, c):
            cg = c0 + c
            l = cg // BT
            bt = cg % BT
            for ft in range(FT):
                pltpu.async_copy(
                    tbs[b].at[pl.ds(FR * ft, FR), pl.ds(0, BC)],
                    out_hbm.at[l, ft, bt], sss[b])

        def drain_store(b):
            for ft in range(FT):
                pltpu.make_async_copy(
                    tbs[b].at[pl.ds(FR * ft, FR), pl.ds(0, BC)],
                    out_hbm.at[0, 0, 0], sss[b]).wait()

        for b in range(4):
            start_gather(b, b)

        def outer(cc, carry):
            for b in range(4):
                c = cc * 4 + b
                wait_gather(b)

                @pl.when(c >= 4)
                def _():
                    drain_store(b)

                @plsc.parallel_loop(0, CHUNK // 16, unroll=4)
                def _(g):
                    parv = par8_v[pl.ds(c * CHUNK + 16 * g, 16)]
                    for ll in range(16):
                        bp = 16 * g + ll
                        p8v = jnp.full((16,), parv[ll], dtype=jnp.int32)
                        bpv = jnp.full((16,), bp, dtype=jnp.int32)
                        for kk in range(4):
                            vals = rows[b][bp, pl.ds(16 * kk, 16)]
                            plsc.store_scatter(
                                tbs[b], [row_ids[kk] ^ p8v, bpv], vals)

                start_store(b, c)

                @pl.when(c + 4 < n_chunks)
                def _():
                    start_gather(b, c + 4)

            return carry

        lax.fori_loop(0, n_chunks // 4, outer, 0)
        for b in range(4):
            drain_store(b)

    return k(xtf, table)


def kernel(x, lut):
    lut_t = lut.T
    tail2 = (lut[NFULL * VB :] * SCALE).reshape(TAIL // 2, 2, 4, 2, 8)
    tail_perm = jnp.stack(
        [tail2[:, 0], tail2[:, 1, :, ::-1, :]], axis=1)
    tail_flat = tail_perm.reshape(TAIL * EMB // 128, 128)
    table_wide = _k1_format_table(lut_t, tail_flat)
    table = table_wide.reshape(VOCAB, EMB)
    xtf = x.T.reshape(16384 * 50)
    out5 = _k2_lookup(xtf, table)
    return jnp.transpose(out5, (2, 4, 0, 1, 3)).reshape(16384, 50, EMB)


# speedup vs baseline: 1.0770x; 1.0770x over previous
import functools

import jax
import jax.numpy as jnp
from jax import lax
from jax.experimental import pallas as pl
from jax.experimental.pallas import tpu as pltpu
from jax.experimental.pallas import tpu_sc as plsc

NC = 2
NS = 16
NW = NC * NS
VOCAB = 1000000
EMB = 64
SCALE = 8.0
VB = 256
NFULL = VOCAB // VB
NJ_BASE = NFULL // NW
TAIL = VOCAB - NFULL * VB
CHUNK = 128


def _k1_format_table(lut_t, tail_flat):
    mesh = plsc.VectorSubcoreMesh(core_axis_name="c", subcore_axis_name="s")

    @functools.partial(
        pl.kernel,
        out_type=jax.ShapeDtypeStruct((VOCAB * EMB // 128, 128), jnp.float32),
        mesh=mesh,
        scratch_types=[
            pltpu.VMEM((EMB, VB), jnp.float32),
            pltpu.VMEM((EMB, VB), jnp.float32),
            pltpu.VMEM((VB // 2, 129), jnp.float32),
            pltpu.VMEM((VB // 2, 129), jnp.float32),
            pltpu.VMEM((TAIL * EMB // 128, 128), jnp.float32),
            pltpu.SemaphoreType.DMA,
            pltpu.SemaphoreType.DMA,
            pltpu.SemaphoreType.DMA,
            pltpu.SemaphoreType.DMA,
        ],
        compiler_params=pltpu.CompilerParams(
            use_tc_tiling_on_sc=True, needs_layout_passes=False),
    )
    def k(lut_hbm, tail_hbm, out_hbm, buf0, buf1, fb0, fb1, tbuf,
          si0, si1, so0, so1):
        wid = lax.axis_index("s") * NC + lax.axis_index("c")
        bufs = (buf0, buf1)
        fbs = (fb0, fb1)
        sis = (si0, si1)
        sos = (so0, so1)

        @pl.when(wid == 0)
        def _():
            pltpu.sync_copy(tail_hbm, tbuf)
            pltpu.sync_copy(
                tbuf, out_hbm.at[pl.ds(NFULL * VB * EMB // 128, TAIL * EMB // 128), :])

        n_j = NJ_BASE + jnp.where(wid < NFULL - NW * NJ_BASE, 1, 0)
        QR = VB // 2
        iota16 = jnp.arange(16, dtype=jnp.int32)
        q_ids = [(iota16 + 16 * mm) // 2 for mm in range(VB // 16)]

        def start_in(b, j):
            bid = wid + NW * j
            pltpu.async_copy(
                lut_hbm.at[:, pl.ds(bid * VB, VB)], bufs[b], sis[b])

        def wait_in(b):
            pltpu.make_async_copy(
                lut_hbm.at[:, pl.ds(0, VB)], bufs[b], sis[b]).wait()

        def start_out(b, j):
            bid = wid + NW * j
            pltpu.async_copy(
                fbs[b].at[:, pl.ds(0, 128)],
                out_hbm.at[pl.ds(bid * QR, QR), :], sos[b])

        def drain_out(b):
            pltpu.make_async_copy(
                fbs[b].at[:, pl.ds(0, 128)],
                out_hbm.at[pl.ds(0, QR), :], sos[b]).wait()

        def transpose(b):
            e8 = (iota16 % 2) * 8
            p64 = (iota16 % 2) * EMB

            @plsc.parallel_loop(0, EMB, unroll=8)
            def _(f):
                rv = (jnp.full((16,), f, dtype=jnp.int32) ^ e8) + p64
                for mm in range(VB // 16):
                    vals = bufs[b][f, pl.ds(16 * mm, 16)] * SCALE
                    plsc.store_scatter(fbs[b], [q_ids[mm], rv], vals)

        start_in(0, 0)
        start_in(1, 1)

        def outer(jj, carry):
            for b in range(2):
                j = jj * 2 + b
                wait_in(b)

                @pl.when(j >= 2)
                def _():
                    drain_out(b)

                transpose(b)
                start_out(b, j)

                @pl.when(j + 2 < n_j)
                def _():
                    start_in(b, j + 2)

            return carry

        lax.fori_loop(0, NJ_BASE // 2, outer, 0)

        @pl.when(n_j > NJ_BASE)
        def _():
            wait_in(0)
            drain_out(0)
            transpose(0)
            start_out(0, NJ_BASE)

        drain_out(0)
        drain_out(1)

    return k(lut_t, tail_flat)


def _k2_lookup(xtf, table):
    L5, FT, BT, FR, BC = 50, 8, 128, 8, 128
    n_chunks = L5 * BT // NW
    mesh = plsc.VectorSubcoreMesh(core_axis_name="c", subcore_axis_name="s")

    @functools.partial(
        pl.kernel,
        out_type=jax.ShapeDtypeStruct((L5, FT, BT, FR, BC), jnp.float32),
        mesh=mesh,
        scratch_types=[
            pltpu.VMEM((n_chunks * CHUNK,), jnp.int32),
            pltpu.VMEM((n_chunks * CHUNK,), jnp.int32),
            pltpu.VMEM((CHUNK, EMB), jnp.float32),
            pltpu.VMEM((CHUNK, EMB), jnp.float32),
            pltpu.VMEM((CHUNK, EMB), jnp.float32),
            pltpu.VMEM((CHUNK, EMB), jnp.float32),
            pltpu.VMEM((EMB, BC + 1), jnp.float32),
            pltpu.VMEM((EMB, BC + 1), jnp.float32),
            pltpu.VMEM((EMB, BC + 1), jnp.float32),
            pltpu.VMEM((EMB, BC + 1), jnp.float32),
            pltpu.SemaphoreType.DMA,
            pltpu.SemaphoreType.DMA,
            pltpu.SemaphoreType.DMA,
            pltpu.SemaphoreType.DMA,
            pltpu.SemaphoreType.DMA,
            pltpu.SemaphoreType.DMA,
            pltpu.SemaphoreType.DMA,
            pltpu.SemaphoreType.DMA,
        ],
        compiler_params=pltpu.CompilerParams(
            use_tc_tiling_on_sc=False, needs_layout_passes=False),
    )
    def k(x_hbm, t_hbm, out_hbm, idx_v, par8_v, r0, r1, r2, r3,
          t0, t1, t2, t3, sg0, sg1, sg2, sg3, ss0, ss1, ss2, ss3):
        wid = lax.axis_index("s") * NC + lax.axis_index("c")
        rows = (r0, r1, r2, r3)
        tbs = (t0, t1, t2, t3)
        sgs = (sg0, sg1, sg2, sg3)
        sss = (ss0, ss1, ss2, ss3)
        c0 = wid * n_chunks

        pltpu.sync_copy(x_hbm.at[pl.ds(c0 * CHUNK, n_chunks * CHUNK)], idx_v)

        @plsc.parallel_loop(0, n_chunks * CHUNK // 16, unroll=8)
        def _(i):
            sl = pl.ds(i * 16, 16)
            par8_v[sl] = (idx_v[sl] & 1) << 3

        row_ids = [jnp.arange(16, dtype=jnp.int32) + 16 * mm for mm in range(4)]

        def start_gather(b, c):
            pltpu.async_copy(
                t_hbm.at[idx_v.at[pl.ds(c * CHUNK, CHUNK)]], rows[b], sgs[b])

        def wait_gather(b):
            pltpu.make_async_copy(
                t_hbm.at[idx_v.at[pl.ds(0, CHUNK)]], rows[b], sgs[b]).wait()

        def start_store(b, c):
            cg = c0 + c
            l = cg // BT
            bt = cg % BT
            for ft in range(FT):
                pltpu.async_copy(
                    tbs[b].at[pl.ds(FR * ft, FR), pl.ds(0, BC)],
                    out_hbm.at[l, ft, bt], sss[b])

        def drain_store(b):
            for ft in range(FT):
                pltpu.make_async_copy(
                    tbs[b].at[pl.ds(FR * ft, FR), pl.ds(0, BC)],
                    out_hbm.at[0, 0, 0], sss[b]).wait()

        for b in range(4):
            start_gather(b, b)

        def outer(cc, carry):
            for b in range(4):
                c = cc * 4 + b
                wait_gather(b)

                @pl.when(c >= 4)
                def _():
                    drain_store(b)

                @plsc.parallel_loop(0, CHUNK // 16, unroll=2)
                def _(g):
                    parv = par8_v[pl.ds(c * CHUNK + 16 * g, 16)]
                    for ll in range(16):
                        bp = 16 * g + ll
                        p8v = jnp.full((16,), parv[ll], dtype=jnp.int32)
                        bpv = jnp.full((16,), bp, dtype=jnp.int32)
                        for kk in range(4):
                            vals = rows[b][bp, pl.ds(16 * kk, 16)]
                            plsc.store_scatter(
                                tbs[b], [row_ids[kk] ^ p8v, bpv], vals)

                start_store(b, c)

                @pl.when(c + 4 < n_chunks)
                def _():
                    start_gather(b, c + 4)

            return carry

        lax.fori_loop(0, n_chunks // 4, outer, 0)
        for b in range(4):
            drain_store(b)

    return k(xtf, table)


def kernel(x, lut):
    lut_t = lut.T
    tail2 = (lut[NFULL * VB :] * SCALE).reshape(TAIL // 2, 2, 4, 2, 8)
    tail_perm = jnp.stack(
        [tail2[:, 0], tail2[:, 1, :, ::-1, :]], axis=1)
    tail_flat = tail_perm.reshape(TAIL * EMB // 128, 128)
    table_wide = _k1_format_table(lut_t, tail_flat)
    table = table_wide.reshape(VOCAB, EMB)
    xtf = x.T.reshape(16384 * 50)
    out5 = _k2_lookup(xtf, table)
    return jnp.transpose(out5, (2, 4, 0, 1, 3)).reshape(16384, 50, EMB)
